# baseline (speedup 1.0000x reference)
import functools

import jax
import jax.numpy as jnp
from jax import lax
from jax.experimental import pallas as pl
from jax.experimental.pallas import tpu as pltpu
from jax.experimental.pallas import tpu_sc as plsc

N, E, D, DE = 10000, 320000, 128, 16
H = 2 * D
F = 2 * D + DE
EPS = 1e-5

NC, NS = 2, 16
NW = NC * NS
NCHUNK = 2
EC = E // NCHUNK
GW = 40
BE = 2000
BN_BLK = 1000
NPC = N // NS
ZROWS = 125

def _dot_t(a, w):
  return lax.dot_general(a, w, (((1,), (1,)), ((), ())),
                         preferred_element_type=jnp.float32)


def _softplus(x):
  return jnp.maximum(x, 0.0) + jnp.log(1.0 + jnp.exp(-jnp.abs(x)))


def _sigmoid(x):
  return 0.5 + 0.5 * jnp.tanh(0.5 * x)


_SC_PARAMS = pltpu.CompilerParams(use_tc_tiling_on_sc=False)


PER = EC // NW
NBUF = 5
GSTEPS = PER // GW


def _sc_gather(atom, idx_s, idx_n, co):
  mesh = plsc.VectorSubcoreMesh(core_axis_name="c", subcore_axis_name="s")

  @functools.partial(
      pl.kernel,
      out_type=[jax.ShapeDtypeStruct((EC, D), jnp.float32),
                jax.ShapeDtypeStruct((EC, D), jnp.float32)],
      mesh=mesh,
      compiler_params=_SC_PARAMS,
      scratch_types=[pltpu.VMEM((PER,), jnp.int32),
                     pltpu.VMEM((PER,), jnp.int32),
                     pltpu.VMEM((NBUF, GW, D), jnp.float32),
                     pltpu.VMEM((NBUF, GW, D), jnp.float32),
                     pltpu.SemaphoreType.DMA((NBUF,)),
                     pltpu.SemaphoreType.DMA((NBUF,)),
                     pltpu.SemaphoreType.DMA])
  def k(x_hbm, is_hbm, in_hbm, os_hbm, on_hbm, ixs_v, ixn_v, bufs, bufn,
        sem_g, sem_w, sem_i):
    c = lax.axis_index("c")
    s = lax.axis_index("s")
    wid = c * NS + s
    base = wid * PER

    pltpu.async_copy(is_hbm.at[0, pl.ds(co + base, PER)], ixs_v, sem_i).wait()
    pltpu.async_copy(in_hbm.at[0, pl.ds(co + base, PER)], ixn_v, sem_i).wait()

    def start_gather(j, b):
      pltpu.async_copy(x_hbm.at[ixs_v.at[pl.ds(j * GW, GW)]], bufs.at[b],
                       sem_g.at[b])
      pltpu.async_copy(x_hbm.at[ixn_v.at[pl.ds(j * GW, GW)]], bufn.at[b],
                       sem_g.at[b])

    def wait_gather(b):
      pltpu.make_async_copy(os_hbm.at[pl.ds(0, GW)], bufs.at[b],
                            sem_g.at[b]).wait()
      pltpu.make_async_copy(on_hbm.at[pl.ds(0, GW)], bufn.at[b],
                            sem_g.at[b]).wait()

    def start_wb(j, b):
      pltpu.async_copy(bufs.at[b], os_hbm.at[pl.ds(base + j * GW, GW)],
                       sem_w.at[b])
      pltpu.async_copy(bufn.at[b], on_hbm.at[pl.ds(base + j * GW, GW)],
                       sem_w.at[b])

    def wait_wb(b):
      pltpu.make_async_copy(bufs.at[b], os_hbm.at[pl.ds(0, GW)],
                            sem_w.at[b]).wait()
      pltpu.make_async_copy(bufn.at[b], on_hbm.at[pl.ds(0, GW)],
                            sem_w.at[b]).wait()

    LOOK = 3
    for t in range(LOOK):
      start_gather(t, t)

    for b in range(NBUF):
      j = b
      wait_gather(b)
      start_wb(j, b)
      t = j + LOOK
      if t >= NBUF:
        wait_wb(t % NBUF)
      start_gather(t, t % NBUF)

    @pl.loop(1, GSTEPS // NBUF - 1)
    def _(jj):
      for b in range(NBUF):
        j = jj * NBUF + b
        bn = (b + LOOK) % NBUF
        wait_gather(b)
        start_wb(j, b)
        wait_wb(bn)
        start_gather(j + LOOK, bn)

    for b in range(NBUF):
      j = GSTEPS - NBUF + b
      wait_gather(b)
      start_wb(j, b)
      if j + LOOK < GSTEPS:
        bn = (b + LOOK) % NBUF
        wait_wb(bn)
        start_gather(j + LOOK, bn)

    for b in range(NBUF):
      wait_wb(b)

  return k(atom, idx_s, idx_n)


def _edge_matmul(gs_ref, gn_ref, nf_ref, w_ref, b_ref):
  w = w_ref[...].astype(jnp.bfloat16)
  x = _dot_t(gs_ref[...].astype(jnp.bfloat16), w[:, :D])
  x = x + _dot_t(gn_ref[...].astype(jnp.bfloat16), w[:, D:2 * D])
  x = x + _dot_t(nf_ref[...].astype(jnp.bfloat16), w[:, 2 * D:])
  return x + b_ref[...]


def _k2_body(gs_ref, gn_ref, nf_ref, w_ref, b_ref, st_ref):
  i = pl.program_id(0)
  x = _edge_matmul(gs_ref, gn_ref, nf_ref, w_ref, b_ref)

  @pl.when(i == 0)
  def _():
    st_ref[...] = jnp.zeros_like(st_ref)

  st_ref[0:1, :] += jnp.sum(x, axis=0, keepdims=True)
  st_ref[1:2, :] += jnp.sum(x * x, axis=0, keepdims=True)


def _tc_matmul_stats(gs, gn, nf, wt, b, cb):
  return pl.pallas_call(
      _k2_body,
      grid=(EC // BE,),
      in_specs=[
          pl.BlockSpec((BE, D), lambda i: (i, 0)),
          pl.BlockSpec((BE, D), lambda i: (i, 0)),
          pl.BlockSpec((BE, DE), lambda i: (i + cb, 0)),
          pl.BlockSpec((H, F), lambda i: (0, 0)),
          pl.BlockSpec((1, H), lambda i: (0, 0)),
      ],
      out_specs=pl.BlockSpec((8, H), lambda i: (0, 0)),
      out_shape=jax.ShapeDtypeStruct((8, H), jnp.float32),
  )(gs, gn, nf, wt, b)


def _k3_body(gs_ref, gn_ref, nf_ref, w_ref, b_ref, sta_ref, stb_ref, g1_ref,
             b1_ref, msg_ref):
  x = _edge_matmul(gs_ref, gn_ref, nf_ref, w_ref, b_ref)
  ssum = sta_ref[0:1, :] + stb_ref[0:1, :]
  ssq = sta_ref[1:2, :] + stb_ref[1:2, :]
  mean = ssum * (1.0 / E)
  var = ssq * (1.0 / E) - mean * mean
  inv = lax.rsqrt(var + EPS)
  x = (x - mean) * (inv * g1_ref[...]) + b1_ref[...]
  msg_ref[...] = _sigmoid(x[:, :D]) * _softplus(x[:, D:])


def _tc_activate(gs, gn, nf, wt, b, sta, stb, g1, b1, cb):
  return pl.pallas_call(
      _k3_body,
      grid=(EC // BE,),
      in_specs=[
          pl.BlockSpec((BE, D), lambda i: (i, 0)),
          pl.BlockSpec((BE, D), lambda i: (i, 0)),
          pl.BlockSpec((BE, DE), lambda i: (i + cb, 0)),
          pl.BlockSpec((H, F), lambda i: (0, 0)),
          pl.BlockSpec((1, H), lambda i: (0, 0)),
          pl.BlockSpec((8, H), lambda i: (0, 0)),
          pl.BlockSpec((8, H), lambda i: (0, 0)),
          pl.BlockSpec((1, H), lambda i: (0, 0)),
          pl.BlockSpec((1, H), lambda i: (0, 0)),
      ],
      out_specs=pl.BlockSpec((BE, D), lambda i: (i, 0)),
      out_shape=jax.ShapeDtypeStruct((EC, D), jnp.float32),
  )(gs, gn, nf, wt, b, sta, stb, g1, b1)


def _sc_segsum(msg, idx_s, co):
  cw = co // GW
  mesh = plsc.VectorSubcoreMesh(core_axis_name="c", subcore_axis_name="s")

  @functools.partial(
      pl.kernel,
      out_type=jax.ShapeDtypeStruct((NC, N, D), jnp.float32),
      mesh=mesh,
      compiler_params=_SC_PARAMS,
      scratch_types=[pltpu.VMEM_SHARED((N, D), jnp.float32),
                     pltpu.VMEM((ZROWS, D), jnp.float32)])
  def k(msg_hbm, idx_hbm, o_hbm, accum, zbuf):
    c = lax.axis_index("c")
    s = lax.axis_index("s")

    @pl.loop(0, ZROWS)
    def _(r):
      @pl.loop(0, D, step=16)
      def _(j):
        zbuf.at[pl.ds(r, 1), pl.ds(j, 16)][...] = jnp.zeros((1, 16),
                                                            jnp.float32)

    @pl.loop(0, NPC, step=ZROWS)
    def _(r0):
      pltpu.sync_copy(zbuf, accum.at[pl.ds(s * NPC + r0, ZROWS)])

    plsc.subcore_barrier()

    def body(msg_v, i_v):
      pltpu.sync_copy(msg_v, accum.at[i_v.at[0]], add=True)

    pltpu.emit_pipeline(
        body,
        grid=(EC // GW,),
        in_specs=[pl.BlockSpec((GW, D), lambda i: (i, 0)),
                  pl.BlockSpec((1, GW), lambda i: (0, i + cw))],
        out_specs=[],
        core_axis_name=("c", "s"),
        dimension_semantics=(pltpu.PARALLEL,),
    )(msg_hbm, idx_hbm)

    plsc.subcore_barrier()
    pltpu.sync_copy(accum.at[pl.ds(s * NPC, NPC)],
                    o_hbm.at[c, pl.ds(s * NPC, NPC)])

  return k(msg, idx_s)


def _k5a_body(pa_ref, pb_ref, st_ref):
  i = pl.program_id(0)

  @pl.when(i == 0)
  def _():
    st_ref[...] = jnp.zeros_like(st_ref)

  sm = (pa_ref[0] + pa_ref[1]) + (pb_ref[0] + pb_ref[1])
  st_ref[0:1, :] += jnp.sum(sm, axis=0, keepdims=True)
  st_ref[1:2, :] += jnp.sum(sm * sm, axis=0, keepdims=True)


def _tc_node_stats(pa, pb):
  return pl.pallas_call(
      _k5a_body,
      grid=(N // BN_BLK,),
      in_specs=[pl.BlockSpec((NC, BN_BLK, D), lambda i: (0, i, 0)),
                pl.BlockSpec((NC, BN_BLK, D), lambda i: (0, i, 0))],
      out_specs=pl.BlockSpec((8, D), lambda i: (0, 0)),
      out_shape=jax.ShapeDtypeStruct((8, D), jnp.float32),
  )(pa, pb)


def _k5b_body(pa_ref, pb_ref, atom_ref, st_ref, g2_ref, b2_ref, o_ref):
  mean = st_ref[0:1, :] * (1.0 / N)
  var = st_ref[1:2, :] * (1.0 / N) - mean * mean
  inv = lax.rsqrt(var + EPS)
  sm = (pa_ref[0] + pa_ref[1]) + (pb_ref[0] + pb_ref[1])
  x = atom_ref[...] + (sm - mean) * (inv * g2_ref[...]) + b2_ref[...]
  o_ref[...] = _softplus(x)


def _tc_output(pa, pb, atom, st2, g2, b2):
  return pl.pallas_call(
      _k5b_body,
      grid=(N // BN_BLK,),
      in_specs=[
          pl.BlockSpec((NC, BN_BLK, D), lambda i: (0, i, 0)),
          pl.BlockSpec((NC, BN_BLK, D), lambda i: (0, i, 0)),
          pl.BlockSpec((BN_BLK, D), lambda i: (i, 0)),
          pl.BlockSpec((8, D), lambda i: (0, 0)),
          pl.BlockSpec((1, D), lambda i: (0, 0)),
          pl.BlockSpec((1, D), lambda i: (0, 0)),
      ],
      out_specs=pl.BlockSpec((BN_BLK, D), lambda i: (i, 0)),
      out_shape=jax.ShapeDtypeStruct((N, D), jnp.float32),
  )(pa, pb, atom, st2, g2, b2)


def kernel(atom_in_fea, nbr_fea, self_fea_idx, nbr_fea_idx, W_fc, b_fc,
           gamma1, beta1, gamma2, beta2):
  idx_s = self_fea_idx.reshape(1, E)
  idx_n = nbr_fea_idx.reshape(1, E)
  b2d = b_fc.reshape(1, H)
  g1 = gamma1.reshape(1, H)
  b1 = beta1.reshape(1, H)

  gs_a, gn_a = _sc_gather(atom_in_fea, idx_s, idx_n, 0)
  gs_b, gn_b = _sc_gather(atom_in_fea, idx_s, idx_n, EC)
  st_a = _tc_matmul_stats(gs_a, gn_a, nbr_fea, W_fc, b2d, 0)
  st_b = _tc_matmul_stats(gs_b, gn_b, nbr_fea, W_fc, b2d, EC // BE)
  msg_a = _tc_activate(gs_a, gn_a, nbr_fea, W_fc, b2d, st_a, st_b, g1, b1, 0)
  msg_b = _tc_activate(gs_b, gn_b, nbr_fea, W_fc, b2d, st_a, st_b, g1, b1,
                       EC // BE)
  pa = _sc_segsum(msg_a, idx_s, 0)
  pb = _sc_segsum(msg_b, idx_s, EC)
  st2 = _tc_node_stats(pa, pb)
  return _tc_output(pa, pb, atom_in_fea, st2, gamma2.reshape(1, D),
                    beta2.reshape(1, D))


# speedup vs baseline: 1.1017x; 1.1017x over previous
import functools

import jax
import jax.numpy as jnp
from jax import lax
from jax.experimental import pallas as pl
from jax.experimental.pallas import tpu as pltpu
from jax.experimental.pallas import tpu_sc as plsc

N, E, D, DE = 10000, 320000, 128, 16
H = 2 * D
F = 2 * D + DE
EPS = 1e-5

NC, NS = 2, 16
NW = NC * NS
ECA = 128000
ECB = E - ECA
GW = 80
BE = 2000
BN_BLK = 1000
NPC = N // NS
ZROWS = 125

def _dot_t(a, w):
  return lax.dot_general(a, w, (((1,), (1,)), ((), ())),
                         preferred_element_type=jnp.float32)


def _softplus(x):
  return jnp.maximum(x, 0.0) + jnp.log(1.0 + jnp.exp(-jnp.abs(x)))


def _sigmoid(x):
  return 0.5 + 0.5 * jnp.tanh(0.5 * x)


_SC_PARAMS = pltpu.CompilerParams(use_tc_tiling_on_sc=False)


NBUF = 5


def _sc_gather(atom, idx_s, idx_n, co, ne):
  per = ne // NW
  gsteps = per // GW
  mesh = plsc.VectorSubcoreMesh(core_axis_name="c", subcore_axis_name="s")

  @functools.partial(
      pl.kernel,
      out_type=[jax.ShapeDtypeStruct((ne, D), jnp.float32),
                jax.ShapeDtypeStruct((ne, D), jnp.float32)],
      mesh=mesh,
      compiler_params=_SC_PARAMS,
      scratch_types=[pltpu.VMEM((per,), jnp.int32),
                     pltpu.VMEM((per,), jnp.int32),
                     pltpu.VMEM((NBUF, GW, D), jnp.float32),
                     pltpu.VMEM((NBUF, GW, D), jnp.float32),
                     pltpu.SemaphoreType.DMA((NBUF,)),
                     pltpu.SemaphoreType.DMA((NBUF,)),
                     pltpu.SemaphoreType.DMA])
  def k(x_hbm, is_hbm, in_hbm, os_hbm, on_hbm, ixs_v, ixn_v, bufs, bufn,
        sem_g, sem_w, sem_i):
    c = lax.axis_index("c")
    s = lax.axis_index("s")
    wid = c * NS + s
    base = wid * per

    pltpu.async_copy(is_hbm.at[0, pl.ds(co + base, per)], ixs_v, sem_i).wait()
    pltpu.async_copy(in_hbm.at[0, pl.ds(co + base, per)], ixn_v, sem_i).wait()

    def start_gather(j, b):
      pltpu.async_copy(x_hbm.at[ixs_v.at[pl.ds(j * GW, GW)]], bufs.at[b],
                       sem_g.at[b])
      pltpu.async_copy(x_hbm.at[ixn_v.at[pl.ds(j * GW, GW)]], bufn.at[b],
                       sem_g.at[b])

    def wait_gather(b):
      pltpu.make_async_copy(os_hbm.at[pl.ds(0, GW)], bufs.at[b],
                            sem_g.at[b]).wait()
      pltpu.make_async_copy(on_hbm.at[pl.ds(0, GW)], bufn.at[b],
                            sem_g.at[b]).wait()

    def start_wb(j, b):
      pltpu.async_copy(bufs.at[b], os_hbm.at[pl.ds(base + j * GW, GW)],
                       sem_w.at[b])
      pltpu.async_copy(bufn.at[b], on_hbm.at[pl.ds(base + j * GW, GW)],
                       sem_w.at[b])

    def wait_wb(b):
      pltpu.make_async_copy(bufs.at[b], os_hbm.at[pl.ds(0, GW)],
                            sem_w.at[b]).wait()
      pltpu.make_async_copy(bufn.at[b], on_hbm.at[pl.ds(0, GW)],
                            sem_w.at[b]).wait()

    for b in range(NBUF):
      start_gather(b, b)

    @pl.loop(0, gsteps // NBUF - 1)
    def _(jj):
      for b in range(NBUF):
        j = jj * NBUF + b
        wait_gather(b)
        start_wb(j, b)
        wait_wb(b)
        start_gather(j + NBUF, b)

    for b in range(NBUF):
      j = gsteps - NBUF + b
      wait_gather(b)
      start_wb(j, b)
      wait_wb(b)

  return k(atom, idx_s, idx_n)


def _edge_matmul(gs_ref, gn_ref, nf_ref, w_ref, b_ref):
  w = w_ref[...].astype(jnp.bfloat16)
  x = _dot_t(gs_ref[...].astype(jnp.bfloat16), w[:, :D])
  x = x + _dot_t(gn_ref[...].astype(jnp.bfloat16), w[:, D:2 * D])
  x = x + _dot_t(nf_ref[...].astype(jnp.bfloat16), w[:, 2 * D:])
  return x + b_ref[...]


def _k2_body(gs_ref, gn_ref, nf_ref, w_ref, b_ref, st_ref):
  i = pl.program_id(0)
  x = _edge_matmul(gs_ref, gn_ref, nf_ref, w_ref, b_ref)

  @pl.when(i == 0)
  def _():
    st_ref[...] = jnp.zeros_like(st_ref)

  st_ref[0:1, :] += jnp.sum(x, axis=0, keepdims=True)
  st_ref[1:2, :] += jnp.sum(x * x, axis=0, keepdims=True)


def _tc_matmul_stats(gs, gn, nf, wt, b, cb, ne):
  return pl.pallas_call(
      _k2_body,
      grid=(ne // BE,),
      in_specs=[
          pl.BlockSpec((BE, D), lambda i: (i, 0)),
          pl.BlockSpec((BE, D), lambda i: (i, 0)),
          pl.BlockSpec((BE, DE), lambda i: (i + cb, 0)),
          pl.BlockSpec((H, F), lambda i: (0, 0)),
          pl.BlockSpec((1, H), lambda i: (0, 0)),
      ],
      out_specs=pl.BlockSpec((8, H), lambda i: (0, 0)),
      out_shape=jax.ShapeDtypeStruct((8, H), jnp.float32),
  )(gs, gn, nf, wt, b)


def _k3_body(gs_ref, gn_ref, nf_ref, w_ref, b_ref, sta_ref, stb_ref, g1_ref,
             b1_ref, msg_ref):
  x = _edge_matmul(gs_ref, gn_ref, nf_ref, w_ref, b_ref)
  ssum = sta_ref[0:1, :] + stb_ref[0:1, :]
  ssq = sta_ref[1:2, :] + stb_ref[1:2, :]
  mean = ssum * (1.0 / E)
  var = ssq * (1.0 / E) - mean * mean
  inv = lax.rsqrt(var + EPS)
  x = (x - mean) * (inv * g1_ref[...]) + b1_ref[...]
  msg_ref[...] = _sigmoid(x[:, :D]) * _softplus(x[:, D:])


def _tc_activate(gs, gn, nf, wt, b, sta, stb, g1, b1, cb, ne):
  return pl.pallas_call(
      _k3_body,
      grid=(ne // BE,),
      in_specs=[
          pl.BlockSpec((BE, D), lambda i: (i, 0)),
          pl.BlockSpec((BE, D), lambda i: (i, 0)),
          pl.BlockSpec((BE, DE), lambda i: (i + cb, 0)),
          pl.BlockSpec((H, F), lambda i: (0, 0)),
          pl.BlockSpec((1, H), lambda i: (0, 0)),
          pl.BlockSpec((8, H), lambda i: (0, 0)),
          pl.BlockSpec((8, H), lambda i: (0, 0)),
          pl.BlockSpec((1, H), lambda i: (0, 0)),
          pl.BlockSpec((1, H), lambda i: (0, 0)),
      ],
      out_specs=pl.BlockSpec((BE, D), lambda i: (i, 0)),
      out_shape=jax.ShapeDtypeStruct((ne, D), jnp.float32),
  )(gs, gn, nf, wt, b, sta, stb, g1, b1)


def _sc_segsum(msg, idx_s, co, ne):
  cw = co // GW
  mesh = plsc.VectorSubcoreMesh(core_axis_name="c", subcore_axis_name="s")

  @functools.partial(
      pl.kernel,
      out_type=jax.ShapeDtypeStruct((NC, N, D), jnp.float32),
      mesh=mesh,
      compiler_params=_SC_PARAMS,
      scratch_types=[pltpu.VMEM_SHARED((N, D), jnp.float32),
                     pltpu.VMEM((ZROWS, D), jnp.float32)])
  def k(msg_hbm, idx_hbm, o_hbm, accum, zbuf):
    c = lax.axis_index("c")
    s = lax.axis_index("s")

    @pl.loop(0, ZROWS)
    def _(r):
      @pl.loop(0, D, step=16)
      def _(j):
        zbuf.at[pl.ds(r, 1), pl.ds(j, 16)][...] = jnp.zeros((1, 16),
                                                            jnp.float32)

    @pl.loop(0, NPC, step=ZROWS)
    def _(r0):
      pltpu.sync_copy(zbuf, accum.at[pl.ds(s * NPC + r0, ZROWS)])

    plsc.subcore_barrier()

    def body(msg_v, i_v):
      pltpu.sync_copy(msg_v, accum.at[i_v.at[0]], add=True)

    pltpu.emit_pipeline(
        body,
        grid=(ne // GW,),
        in_specs=[pl.BlockSpec((GW, D), lambda i: (i, 0)),
                  pl.BlockSpec((1, GW), lambda i: (0, i + cw))],
        out_specs=[],
        core_axis_name=("c", "s"),
        dimension_semantics=(pltpu.PARALLEL,),
    )(msg_hbm, idx_hbm)

    plsc.subcore_barrier()
    pltpu.sync_copy(accum.at[pl.ds(s * NPC, NPC)],
                    o_hbm.at[c, pl.ds(s * NPC, NPC)])

  return k(msg, idx_s)


def _k5a_body(pa_ref, pb_ref, st_ref):
  i = pl.program_id(0)

  @pl.when(i == 0)
  def _():
    st_ref[...] = jnp.zeros_like(st_ref)

  sm = (pa_ref[0] + pa_ref[1]) + (pb_ref[0] + pb_ref[1])
  st_ref[0:1, :] += jnp.sum(sm, axis=0, keepdims=True)
  st_ref[1:2, :] += jnp.sum(sm * sm, axis=0, keepdims=True)


def _tc_node_stats(pa, pb):
  return pl.pallas_call(
      _k5a_body,
      grid=(N // BN_BLK,),
      in_specs=[pl.BlockSpec((NC, BN_BLK, D), lambda i: (0, i, 0)),
                pl.BlockSpec((NC, BN_BLK, D), lambda i: (0, i, 0))],
      out_specs=pl.BlockSpec((8, D), lambda i: (0, 0)),
      out_shape=jax.ShapeDtypeStruct((8, D), jnp.float32),
  )(pa, pb)


def _k5b_body(pa_ref, pb_ref, atom_ref, st_ref, g2_ref, b2_ref, o_ref):
  mean = st_ref[0:1, :] * (1.0 / N)
  var = st_ref[1:2, :] * (1.0 / N) - mean * mean
  inv = lax.rsqrt(var + EPS)
  sm = (pa_ref[0] + pa_ref[1]) + (pb_ref[0] + pb_ref[1])
  x = atom_ref[...] + (sm - mean) * (inv * g2_ref[...]) + b2_ref[...]
  o_ref[...] = _softplus(x)


def _tc_output(pa, pb, atom, st2, g2, b2):
  return pl.pallas_call(
      _k5b_body,
      grid=(N // BN_BLK,),
      in_specs=[
          pl.BlockSpec((NC, BN_BLK, D), lambda i: (0, i, 0)),
          pl.BlockSpec((NC, BN_BLK, D), lambda i: (0, i, 0)),
          pl.BlockSpec((BN_BLK, D), lambda i: (i, 0)),
          pl.BlockSpec((8, D), lambda i: (0, 0)),
          pl.BlockSpec((1, D), lambda i: (0, 0)),
          pl.BlockSpec((1, D), lambda i: (0, 0)),
      ],
      out_specs=pl.BlockSpec((BN_BLK, D), lambda i: (i, 0)),
      out_shape=jax.ShapeDtypeStruct((N, D), jnp.float32),
  )(pa, pb, atom, st2, g2, b2)


def kernel(atom_in_fea, nbr_fea, self_fea_idx, nbr_fea_idx, W_fc, b_fc,
           gamma1, beta1, gamma2, beta2):
  idx_s = self_fea_idx.reshape(1, E)
  idx_n = nbr_fea_idx.reshape(1, E)
  b2d = b_fc.reshape(1, H)
  g1 = gamma1.reshape(1, H)
  b1 = beta1.reshape(1, H)

  gs_a, gn_a = _sc_gather(atom_in_fea, idx_s, idx_n, 0, ECA)
  gs_b, gn_b = _sc_gather(atom_in_fea, idx_s, idx_n, ECA, ECB)
  st_a = _tc_matmul_stats(gs_a, gn_a, nbr_fea, W_fc, b2d, 0, ECA)
  st_b = _tc_matmul_stats(gs_b, gn_b, nbr_fea, W_fc, b2d, ECA // BE, ECB)
  msg_a = _tc_activate(gs_a, gn_a, nbr_fea, W_fc, b2d, st_a, st_b, g1, b1,
                       0, ECA)
  msg_b = _tc_activate(gs_b, gn_b, nbr_fea, W_fc, b2d, st_a, st_b, g1, b1,
                       ECA // BE, ECB)
  pa = _sc_segsum(msg_a, idx_s, 0, ECA)
  pb = _sc_segsum(msg_b, idx_s, ECA, ECB)
  st2 = _tc_node_stats(pa, pb)
  return _tc_output(pa, pb, atom_in_fea, st2, gamma2.reshape(1, D),
                    beta2.reshape(1, D))
